# baseline (speedup 1.0000x reference)
import functools

import jax
import jax.numpy as jnp
from jax import lax
from jax.experimental import pallas as pl
from jax.experimental.pallas import tpu as pltpu
from jax.experimental.pallas import tpu_sc as plsc

N_ROWS = 10000
D = 128
NC, NS, L = 2, 16, 16
NW = NC * NS
CHUNK = 128
NCH0, NCH1 = 40, 118
ROWS_PAD = 10240
RPT = ROWS_PAD // NS
EPS = 1e-6


def _mesh():
    return plsc.VectorSubcoreMesh(
        core_axis_name="c", subcore_axis_name="s", num_cores=NC, num_subcores=NS
    )


def _linear_body(x_ref, w_ref, b_ref, o_ref):
    val = lax.dot_general(
        x_ref[...], w_ref[...], (((1,), (1,)), ((), ())),
        preferred_element_type=jnp.float32,
    ) + b_ref[...]
    pid = pl.program_id(0)
    o_ref[...] = jnp.where(pid < N_ROWS // 80, val, 0.0)


def _linear(x, W, b):
    grid = ROWS_PAD // 80
    return pl.pallas_call(
        _linear_body,
        grid=(grid,),
        in_specs=[
            pl.BlockSpec((80, D), lambda i: (jnp.minimum(i, N_ROWS // 80 - 1), 0)),
            pl.BlockSpec((D, D), lambda i: (0, 0)),
            pl.BlockSpec((1, D), lambda i: (0, 0)),
        ],
        out_specs=pl.BlockSpec((80, D), lambda i: (i, 0)),
        out_shape=jax.ShapeDtypeStruct((ROWS_PAD, D), jnp.float32),
    )(x, W, b.reshape(1, D))


def _agg_body(do_hist, src, idx_hbm, zrow, z1, *refs):
    if do_hist:
        (out_acc, out_hb, out_hd, acc, hb, hd,
         ix0, ix1, r0, r1, ones_v, sg0, sg1) = refs
    else:
        out_acc, acc, ix0, ix1, r0, r1, sg0, sg1 = refs
    cid = lax.axis_index("c")
    sid = lax.axis_index("s")
    nch = jnp.where(cid == 0, NCH0, NCH1)
    base = jnp.where(cid == 0, sid * NCH0, NS * NCH0 + sid * NCH1)
    ix = (ix0, ix1)
    rr = (r0, r1)
    sg = (sg0, sg1)

    pltpu.sync_copy(zrow.at[pl.ds(sid * RPT, RPT)], acc.at[pl.ds(sid * RPT, RPT)])
    if do_hist:
        pltpu.sync_copy(z1.at[pl.ds(sid * RPT, RPT)], hb.at[pl.ds(sid * RPT, RPT)])
        pltpu.sync_copy(z1.at[pl.ds(sid * RPT, RPT)], hd.at[pl.ds(sid * RPT, RPT)])
        for k in range(CHUNK // L):
            ones_v[pl.ds(k * L, L)] = jnp.full((L,), 1.0, jnp.float32)
    plsc.subcore_barrier()

    pltpu.sync_copy(idx_hbm.at[base], ix0)
    pltpu.async_copy(src.at[ix0.at[0]], r0, sg0)

    def pair(jj, carry):
        for b in range(2):
            j = 2 * jj + b
            nb = 1 - b

            @pl.when(j + 1 < nch)
            def _prefetch():
                pltpu.sync_copy(idx_hbm.at[base + j + 1], ix[nb])
                pltpu.async_copy(src.at[ix[nb].at[0]], rr[nb], sg[nb])

            pltpu.make_async_copy(src.at[ix[b].at[0]], rr[b], sg[b]).wait()
            pltpu.sync_copy(rr[b], acc.at[ix[b].at[1]], add=True)
            if do_hist:
                pltpu.sync_copy(ones_v, hb.at[ix[b].at[1]], add=True)
                pltpu.sync_copy(ones_v, hd.at[ix[b].at[0]], add=True)
        return carry

    lax.fori_loop(0, nch // 2, pair, 0)
    plsc.subcore_barrier()

    sl = pl.ds(sid * RPT, RPT)
    pltpu.sync_copy(acc.at[sl], out_acc.at[cid, sl])
    if do_hist:
        pltpu.sync_copy(hb.at[sl], out_hb.at[cid, sl])
        pltpu.sync_copy(hd.at[sl], out_hd.at[cid, sl])


def _aggregate(src, idx_hbm, do_hist):
    out_type = [jax.ShapeDtypeStruct((NC, ROWS_PAD, D), jnp.float32)]
    scratch = [
        pltpu.VMEM_SHARED((ROWS_PAD, D), jnp.float32),
    ]
    if do_hist:
        out_type += [
            jax.ShapeDtypeStruct((NC, ROWS_PAD), jnp.float32),
            jax.ShapeDtypeStruct((NC, ROWS_PAD), jnp.float32),
        ]
        scratch += [
            pltpu.VMEM_SHARED((ROWS_PAD,), jnp.float32),
            pltpu.VMEM_SHARED((ROWS_PAD,), jnp.float32),
        ]
    scratch += [
        pltpu.VMEM((2, CHUNK), jnp.int32),
        pltpu.VMEM((2, CHUNK), jnp.int32),
        pltpu.VMEM((CHUNK, D), jnp.float32),
        pltpu.VMEM((CHUNK, D), jnp.float32),
    ]
    if do_hist:
        scratch.append(pltpu.VMEM((CHUNK,), jnp.float32))
    scratch += [pltpu.SemaphoreType.DMA, pltpu.SemaphoreType.DMA]

    zrow = jnp.zeros((ROWS_PAD, D), jnp.float32)
    z1 = jnp.zeros((ROWS_PAD,), jnp.float32)
    fn = pl.kernel(
        functools.partial(_agg_body, do_hist),
        out_type=out_type,
        mesh=_mesh(),
        scratch_types=scratch,
    )
    return fn(src, idx_hbm, zrow, z1)


def _scale_body(p_ref, h_ref, w_ref, o_ref):
    s = w_ref[...] / (h_ref[0] + h_ref[1] + EPS)
    o_ref[...] = (p_ref[0] + p_ref[1]) * s


def _combine_scale(parts, hist, w_num, out_rows):
    R = 80
    hist3 = hist.reshape(NC, ROWS_PAD, 1)
    return pl.pallas_call(
        _scale_body,
        grid=(out_rows // R,),
        in_specs=[
            pl.BlockSpec((NC, R, D), lambda i: (0, i, 0)),
            pl.BlockSpec((NC, R, 1), lambda i: (0, i, 0)),
            pl.BlockSpec((R, 1), lambda i: (i, 0)),
        ],
        out_specs=pl.BlockSpec((R, D), lambda i: (i, 0)),
        out_shape=jax.ShapeDtypeStruct((out_rows, D), jnp.float32),
    )(parts, hist3, w_num.reshape(ROWS_PAD, 1)[:out_rows])


def kernel(x, hyperedge_index, W, b, hyperedge_weight):
    node_idx = hyperedge_index[0]
    edge_idx = hyperedge_index[1]
    n_inc = node_idx.shape[0]
    n_chunks = NS * (NCH0 + NCH1)
    assert n_chunks * CHUNK >= n_inc
    pad = n_chunks * CHUNK - n_inc
    nid = jnp.concatenate([node_idx, jnp.full((pad,), N_ROWS, jnp.int32)])
    eid = jnp.concatenate([edge_idx, jnp.full((pad,), N_ROWS, jnp.int32)])
    nid = nid.reshape(n_chunks, CHUNK)
    eid = eid.reshape(n_chunks, CHUNK)
    idx_ne = jnp.stack([nid, eid], axis=1)
    idx_en = jnp.stack([eid, nid], axis=1)

    x_t = _linear(x, W, b)

    he_parts, hist_b, hist_d = _aggregate(x_t, idx_ne, do_hist=True)
    w_pad = jnp.pad(hyperedge_weight, (0, ROWS_PAD - N_ROWS))
    he_scaled = _combine_scale(he_parts, hist_b, w_pad, ROWS_PAD)

    (node_parts,) = _aggregate(he_scaled, idx_en, do_hist=False)
    ones = jnp.ones((ROWS_PAD,), jnp.float32)
    out = _combine_scale(node_parts, hist_d, ones, N_ROWS)
    return out


# speedup vs baseline: 1.1222x; 1.1222x over previous
import functools

import jax
import jax.numpy as jnp
from jax import lax
from jax.experimental import pallas as pl
from jax.experimental.pallas import tpu as pltpu
from jax.experimental.pallas import tpu_sc as plsc

N_ROWS = 10000
D = 128
NC, NS, L = 2, 16, 16
NW = NC * NS
CHUNK = 64
NCH0, NCH1 = 104, 212
ROWS_PAD = 10240
RPT = ROWS_PAD // NS
EPS = 1e-6


def _mesh():
    return plsc.VectorSubcoreMesh(
        core_axis_name="c", subcore_axis_name="s", num_cores=NC, num_subcores=NS
    )


def _linear_body(x_ref, w_ref, b_ref, o_ref):
    val = lax.dot_general(
        x_ref[...], w_ref[...], (((1,), (1,)), ((), ())),
        preferred_element_type=jnp.float32,
    ) + b_ref[...]
    pid = pl.program_id(0)
    o_ref[...] = jnp.where(pid < N_ROWS // 80, val, 0.0)


def _linear(x, W, b):
    grid = ROWS_PAD // 80
    return pl.pallas_call(
        _linear_body,
        grid=(grid,),
        in_specs=[
            pl.BlockSpec((80, D), lambda i: (jnp.minimum(i, N_ROWS // 80 - 1), 0)),
            pl.BlockSpec((D, D), lambda i: (0, 0)),
            pl.BlockSpec((1, D), lambda i: (0, 0)),
        ],
        out_specs=pl.BlockSpec((80, D), lambda i: (i, 0)),
        out_shape=jax.ShapeDtypeStruct((ROWS_PAD, D), jnp.float32),
    )(x, W, b.reshape(1, D))


def _agg_body(do_hist, src, idx_hbm, zrow, z1, *refs):
    if do_hist:
        out_acc, out_hb, out_hd, acc, hb, hd = refs[:6]
        rest = refs[6:]
    else:
        out_acc, acc = refs[:2]
        rest = refs[2:]
    ix = rest[0:4]
    rr = rest[4:8]
    rest = rest[8:]
    if do_hist:
        ones_v = rest[0]
        rest = rest[1:]
    si = rest[0:4]
    sg = rest[4:8]
    ss = rest[8:12]

    cid = lax.axis_index("c")
    sid = lax.axis_index("s")
    nch = jnp.where(cid == 0, NCH0, NCH1)
    base = jnp.where(cid == 0, sid * NCH0, NS * NCH0 + sid * NCH1)

    pltpu.sync_copy(zrow.at[pl.ds(sid * RPT, RPT)], acc.at[pl.ds(sid * RPT, RPT)])
    if do_hist:
        pltpu.sync_copy(z1.at[pl.ds(sid * RPT, RPT)], hb.at[pl.ds(sid * RPT, RPT)])
        pltpu.sync_copy(z1.at[pl.ds(sid * RPT, RPT)], hd.at[pl.ds(sid * RPT, RPT)])
        for k in range(CHUNK // L):
            ones_v[pl.ds(k * L, L)] = jnp.full((L,), 1.0, jnp.float32)
    plsc.subcore_barrier()

    def scat_descs(b):
        d = [pltpu.make_async_copy(rr[b], acc.at[ix[b].at[1]], ss[b])]
        if do_hist:
            d.append(pltpu.make_async_copy(ones_v, hb.at[ix[b].at[1]], ss[b]))
            d.append(pltpu.make_async_copy(ones_v, hd.at[ix[b].at[0]], ss[b]))
        return d

    pltpu.async_copy(idx_hbm.at[base], ix[0], si[0])
    pltpu.async_copy(idx_hbm.at[base + 1], ix[1], si[1])
    pltpu.make_async_copy(idx_hbm.at[base], ix[0], si[0]).wait()
    pltpu.async_copy(src.at[ix[0].at[0]], rr[0], sg[0])

    def quad(jj, carry):
        for b in range(4):
            j = 4 * jj + b
            b1, b2 = (b + 1) % 4, (b + 2) % 4

            @pl.when(j >= 2)
            def _drain():
                for d in scat_descs(b2):
                    d.wait()

            @pl.when(j + 2 < nch)
            def _idx():
                pltpu.async_copy(idx_hbm.at[base + j + 2], ix[b2], si[b2])

            @pl.when(j + 1 < nch)
            def _gath():
                pltpu.make_async_copy(idx_hbm.at[base + j + 1], ix[b1], si[b1]).wait()
                pltpu.async_copy(src.at[ix[b1].at[0]], rr[b1], sg[b1])

            pltpu.make_async_copy(src.at[ix[b].at[0]], rr[b], sg[b]).wait()
            pltpu.async_copy(rr[b], acc.at[ix[b].at[1]], ss[b], add=True)
            if do_hist:
                pltpu.async_copy(ones_v, hb.at[ix[b].at[1]], ss[b], add=True)
                pltpu.async_copy(ones_v, hd.at[ix[b].at[0]], ss[b], add=True)
        return carry

    lax.fori_loop(0, nch // 4, quad, 0)
    for d in scat_descs(2):
        d.wait()
    for d in scat_descs(3):
        d.wait()
    plsc.subcore_barrier()

    sl = pl.ds(sid * RPT, RPT)
    pltpu.sync_copy(acc.at[sl], out_acc.at[cid, sl])
    if do_hist:
        pltpu.sync_copy(hb.at[sl], out_hb.at[cid, sl])
        pltpu.sync_copy(hd.at[sl], out_hd.at[cid, sl])


def _aggregate(src, idx_hbm, do_hist):
    out_type = [jax.ShapeDtypeStruct((NC, ROWS_PAD, D), jnp.float32)]
    scratch = [
        pltpu.VMEM_SHARED((ROWS_PAD, D), jnp.float32),
    ]
    if do_hist:
        out_type += [
            jax.ShapeDtypeStruct((NC, ROWS_PAD), jnp.float32),
            jax.ShapeDtypeStruct((NC, ROWS_PAD), jnp.float32),
        ]
        scratch += [
            pltpu.VMEM_SHARED((ROWS_PAD,), jnp.float32),
            pltpu.VMEM_SHARED((ROWS_PAD,), jnp.float32),
        ]
    scratch += [pltpu.VMEM((2, CHUNK), jnp.int32) for _ in range(4)]
    scratch += [pltpu.VMEM((CHUNK, D), jnp.float32) for _ in range(4)]
    if do_hist:
        scratch.append(pltpu.VMEM((CHUNK,), jnp.float32))
    scratch += [pltpu.SemaphoreType.DMA for _ in range(12)]

    zrow = jnp.zeros((ROWS_PAD, D), jnp.float32)
    z1 = jnp.zeros((ROWS_PAD,), jnp.float32)
    fn = pl.kernel(
        functools.partial(_agg_body, do_hist),
        out_type=out_type,
        mesh=_mesh(),
        scratch_types=scratch,
    )
    return fn(src, idx_hbm, zrow, z1)


def _scale_body(p_ref, h_ref, w_ref, o_ref):
    s = w_ref[...] / (h_ref[0] + h_ref[1] + EPS)
    o_ref[...] = (p_ref[0] + p_ref[1]) * s


def _combine_scale(parts, hist, w_num, out_rows):
    R = 80
    hist3 = hist.reshape(NC, ROWS_PAD, 1)
    return pl.pallas_call(
        _scale_body,
        grid=(out_rows // R,),
        in_specs=[
            pl.BlockSpec((NC, R, D), lambda i: (0, i, 0)),
            pl.BlockSpec((NC, R, 1), lambda i: (0, i, 0)),
            pl.BlockSpec((R, 1), lambda i: (i, 0)),
        ],
        out_specs=pl.BlockSpec((R, D), lambda i: (i, 0)),
        out_shape=jax.ShapeDtypeStruct((out_rows, D), jnp.float32),
    )(parts, hist3, w_num.reshape(ROWS_PAD, 1)[:out_rows])


def kernel(x, hyperedge_index, W, b, hyperedge_weight):
    node_idx = hyperedge_index[0]
    edge_idx = hyperedge_index[1]
    n_inc = node_idx.shape[0]
    n_chunks = NS * (NCH0 + NCH1)
    assert n_chunks * CHUNK >= n_inc
    pad = n_chunks * CHUNK - n_inc
    nid = jnp.concatenate([node_idx, jnp.full((pad,), N_ROWS, jnp.int32)])
    eid = jnp.concatenate([edge_idx, jnp.full((pad,), N_ROWS, jnp.int32)])
    nid = nid.reshape(n_chunks, CHUNK)
    eid = eid.reshape(n_chunks, CHUNK)
    idx_ne = jnp.stack([nid, eid], axis=1)
    idx_en = jnp.stack([eid, nid], axis=1)

    x_t = _linear(x, W, b)

    he_parts, hist_b, hist_d = _aggregate(x_t, idx_ne, do_hist=True)
    w_pad = jnp.pad(hyperedge_weight, (0, ROWS_PAD - N_ROWS))
    he_scaled = _combine_scale(he_parts, hist_b, w_pad, ROWS_PAD)

    (node_parts,) = _aggregate(he_scaled, idx_en, do_hist=False)
    ones = jnp.ones((ROWS_PAD,), jnp.float32)
    out = _combine_scale(node_parts, hist_d, ones, N_ROWS)
    return out
